# Initial kernel scaffold; baseline (speedup 1.0000x reference)
import jax
import jax.numpy as jnp
from jax.experimental import pallas as pl


def kernel(x, edge_index, batch, edge_mask, W1, b1, W2, b2, Wfc, bfc):
    raise NotImplementedError("write your pallas kernel here")


# speedup vs baseline: 12.5482x; 12.5482x over previous
import functools

import jax
import jax.numpy as jnp
from jax import lax
from jax.experimental import pallas as pl
from jax.experimental.pallas import tpu as pltpu
from jax.experimental.pallas import tpu_sc as plsc

N = 10000
E = 320000
D = 128
G = 64
C = 10

NC = 2
NS = 16
NW = NC * NS
L = 16

EPT = E // NW
DEG_BLK = 2000
CH = 128
NCHUNKS = E // CH
STRIPE = 640
LAST_STRIPE = N - (NS - 1) * STRIPE

_mesh = plsc.VectorSubcoreMesh(core_axis_name="c", subcore_axis_name="s",
                               num_cores=NC)


@functools.partial(
    pl.kernel,
    out_type=jax.ShapeDtypeStruct((NC, N), jnp.float32),
    mesh=_mesh,
    scratch_types=[
        pltpu.VMEM((CH,), jnp.int32),
        pltpu.VMEM((CH,), jnp.float32),
        pltpu.VMEM_SHARED((N,), jnp.float32),
    ],
)
def _sc_degree(col_hbm, mask_hbm, zeros1d_hbm, out_hbm, colbuf, maskbuf, deg):
    cid = lax.axis_index("c")
    sid = lax.axis_index("s")
    wid = sid * NC + cid

    @pl.when(sid == 0)
    def _():
        pltpu.sync_copy(zeros1d_hbm, deg)

    plsc.subcore_barrier()

    def chunk_body(t, carry):
        c = wid + NW * t

        @pl.when(c < NCHUNKS)
        def _():
            base = c * CH
            pltpu.sync_copy(col_hbm.at[pl.ds(base, CH)], colbuf)
            pltpu.sync_copy(mask_hbm.at[pl.ds(base, CH)], maskbuf)
            pltpu.sync_copy(maskbuf, deg.at[colbuf], add=True)

        return carry

    lax.fori_loop(0, (NCHUNKS + NW - 1) // NW, chunk_body, None)
    plsc.subcore_barrier()

    @pl.when(sid == 0)
    def _():
        pltpu.sync_copy(deg, out_hbm.at[cid])


@functools.partial(
    pl.kernel,
    out_type=jax.ShapeDtypeStruct((NC, N, D), jnp.float32),
    mesh=_mesh,
    scratch_types=[
        pltpu.VMEM((CH,), jnp.int32),
        pltpu.VMEM((CH,), jnp.int32),
        pltpu.VMEM((CH,), jnp.float32),
        pltpu.VMEM((CH, D), jnp.float32),
        pltpu.VMEM_SHARED((N, D), jnp.float32),
        pltpu.SemaphoreType.DMA,
    ],
)
def _sc_aggregate(xs_hbm, row_hbm, col_hbm, mask_hbm, zeros_hbm, out_hbm,
                  rowbuf, colbuf, maskbuf, rows, acc, sem):
    cid = lax.axis_index("c")
    sid = lax.axis_index("s")
    wid = sid * NC + cid

    @pl.when(sid < NS - 1)
    def _():
        pltpu.sync_copy(zeros_hbm, acc.at[pl.ds(sid * STRIPE, STRIPE)])

    @pl.when(sid == NS - 1)
    def _():
        pltpu.sync_copy(zeros_hbm.at[pl.ds(0, LAST_STRIPE)],
                        acc.at[pl.ds((NS - 1) * STRIPE, LAST_STRIPE)])

    plsc.subcore_barrier()

    def chunk_body(t, carry):
        c = wid + NW * t

        @pl.when(c < NCHUNKS)
        def _():
            base = c * CH
            pltpu.sync_copy(row_hbm.at[pl.ds(base, CH)], rowbuf)
            pltpu.sync_copy(col_hbm.at[pl.ds(base, CH)], colbuf)
            pltpu.sync_copy(mask_hbm.at[pl.ds(base, CH)], maskbuf)
            pltpu.async_copy(xs_hbm.at[rowbuf], rows, sem).wait()

            def scale_body(i, c2):
                wv = maskbuf[pl.ds(i * L, L)]
                for lane in range(L):
                    w = wv[lane]
                    k = i * L + lane
                    for j in range(D // L):
                        rows[k, pl.ds(j * L, L)] = rows[k, pl.ds(j * L, L)] * w
                return c2

            lax.fori_loop(0, CH // L, scale_body, None)
            pltpu.sync_copy(rows, acc.at[colbuf], add=True)

        return carry

    lax.fori_loop(0, (NCHUNKS + NW - 1) // NW, chunk_body, None)
    plsc.subcore_barrier()

    @pl.when(sid < NS - 1)
    def _():
        pltpu.sync_copy(acc.at[pl.ds(sid * STRIPE, STRIPE)],
                        out_hbm.at[cid, pl.ds(sid * STRIPE, STRIPE)])

    @pl.when(sid == NS - 1)
    def _():
        pltpu.sync_copy(acc.at[pl.ds((NS - 1) * STRIPE, LAST_STRIPE)],
                        out_hbm.at[cid, pl.ds((NS - 1) * STRIPE, LAST_STRIPE)])


def _tc1_body(degp_ref, x_ref, w1_ref, dinv_ref, xs_ref):
    deg = degp_ref[0] + degp_ref[1] + 1.0
    dinv = jnp.where(deg > 0.0, lax.rsqrt(deg), 0.0)
    dinv = dinv[:, None]
    xw = jnp.dot(x_ref[...], w1_ref[...], preferred_element_type=jnp.float32)
    dinv_ref[...] = dinv
    xs_ref[...] = xw * dinv


def _tc2_body(s_ref, xs_ref, dinv_ref, b_ref, w2_ref, out_ref):
    dinv = dinv_ref[...]
    h = (s_ref[0] + s_ref[1] + xs_ref[...]) * dinv + b_ref[...]
    h = jnp.maximum(h, 0.0)
    xw2 = jnp.dot(h, w2_ref[...], preferred_element_type=jnp.float32)
    out_ref[...] = xw2 * dinv


def _tc3_body(s_ref, xs_ref, dinv_ref, b_ref, batch_ref, wfc_ref, bfc_ref,
              out_ref):
    dinv = dinv_ref[...]
    h = (s_ref[0] + s_ref[1] + xs_ref[...]) * dinv + b_ref[...]
    h = jnp.maximum(h, 0.0)
    gids = lax.broadcasted_iota(jnp.int32, (N, G), 1)
    onehot = (batch_ref[...] == gids).astype(jnp.float32)
    pooled = lax.dot_general(onehot, h, (((0,), (0,)), ((), ())),
                             preferred_element_type=jnp.float32)
    logits = jnp.dot(pooled, wfc_ref[...],
                     preferred_element_type=jnp.float32) + bfc_ref[...]
    m = jnp.max(logits, axis=1, keepdims=True)
    lse = m + jnp.log(jnp.sum(jnp.exp(logits - m), axis=1, keepdims=True))
    out_ref[...] = logits - lse


_tc1 = pl.pallas_call(
    _tc1_body,
    out_shape=(jax.ShapeDtypeStruct((N, 1), jnp.float32),
               jax.ShapeDtypeStruct((N, D), jnp.float32)),
)

_tc2 = pl.pallas_call(
    _tc2_body,
    out_shape=jax.ShapeDtypeStruct((N, D), jnp.float32),
)

_tc3 = pl.pallas_call(
    _tc3_body,
    out_shape=jax.ShapeDtypeStruct((G, C), jnp.float32),
)


def kernel(x, edge_index, batch, edge_mask, W1, b1, W2, b2, Wfc, bfc):
    row = edge_index[0]
    col = edge_index[1]
    zeros = jnp.zeros((STRIPE, D), jnp.float32)
    zeros1d = jnp.zeros((N,), jnp.float32)
    batch2d = batch.reshape(N, 1).astype(jnp.int32)

    degp = _sc_degree(col, edge_mask, zeros1d)
    dinv, xs1 = _tc1(degp, x, W1)
    s1 = _sc_aggregate(xs1, row, col, edge_mask, zeros)
    xs2 = _tc2(s1, xs1, dinv, b1.reshape(1, D), W2)
    s2 = _sc_aggregate(xs2, row, col, edge_mask, zeros)
    return _tc3(s2, xs2, dinv, b2.reshape(1, D), batch2d, Wfc, bfc)
